# Initial kernel scaffold; baseline (speedup 1.0000x reference)
import jax
import jax.numpy as jnp
from jax.experimental import pallas as pl


def kernel(x, token_embedding, pe):
    raise NotImplementedError("write your pallas kernel here")


# speedup vs baseline: 9.8171x; 9.8171x over previous
import functools

import jax
import jax.numpy as jnp
from jax import lax
from jax.experimental import pallas as pl
from jax.experimental.pallas import tpu as pltpu
from jax.experimental.pallas import tpu_sc as plsc

D = 256
HALF = 128
NB = 8
BATCH = 4096
SEQ = 64
NROWS = BATCH * SEQ
TAB = NB * NB * NB

_info = plsc.get_sparse_core_info()
_NC, _NS = _info.num_cores, _info.num_subcores
_NW = _NC * _NS
_BPW = BATCH // _NW
_CHUNK_B = 2
_CHUNK_R = _CHUNK_B * SEQ
_NCHUNK = _BPW // _CHUNK_B
_XPW = _BPW * 3 * SEQ


def _table_body(te_ref, pe_ref, c_ref):
    te8 = te_ref[...]
    pe = pe_ref[...]
    first = jnp.broadcast_to(pe[:, None, :], (NB, NB, HALF)).reshape(NB * NB, HALF)
    second = jnp.broadcast_to(pe[None, :, :], (NB, NB, HALF)).reshape(NB * NB, HALF)
    pos = jnp.concatenate([first, second], axis=-1)
    tok = jnp.broadcast_to(te8[:, None, :] * 16.0, (NB, NB * NB, D)).reshape(TAB, D)
    posr = jnp.broadcast_to(pos[None, :, :], (NB, NB * NB, D)).reshape(TAB, D)
    c_ref[...] = tok + posr


_mesh = plsc.VectorSubcoreMesh(core_axis_name="c", subcore_axis_name="s")


@functools.partial(
    pl.kernel,
    mesh=_mesh,
    out_type=jax.ShapeDtypeStruct((NROWS, D), jnp.float32),
    scratch_types=[
        pltpu.VMEM((_XPW,), jnp.int32),
        pltpu.VMEM((_CHUNK_R,), jnp.int32),
        pltpu.VMEM((_CHUNK_R, D), jnp.float32),
        pltpu.SemaphoreType.DMA,
    ],
)
def _sc_gather(xf_hbm, tab_hbm, out_hbm, xv, idxv, rowsv, sem):
    wid = lax.axis_index("s") * _NC + lax.axis_index("c")
    pltpu.sync_copy(xf_hbm.at[pl.ds(wid * _XPW, _XPW)], xv)
    rbase = wid * (_BPW * SEQ)

    def chunk(c, carry):
        for j in range(_CHUNK_B):
            boff = (c * _CHUNK_B + j) * (3 * SEQ)
            for k in range(SEQ // 16):
                t = xv[pl.ds(boff + k * 16, 16)]
                p1 = xv[pl.ds(boff + SEQ + k * 16, 16)]
                p2 = xv[pl.ds(boff + 2 * SEQ + k * 16, 16)]
                idxv[pl.ds(j * SEQ + k * 16, 16)] = t * (NB * NB) + p1 * NB + p2
        pltpu.async_copy(tab_hbm.at[idxv], rowsv, sem).wait()
        pltpu.sync_copy(rowsv, out_hbm.at[pl.ds(rbase + c * _CHUNK_R, _CHUNK_R)])
        return carry

    lax.fori_loop(0, _NCHUNK, chunk, 0)


def kernel(x, token_embedding, pe):
    tab = pl.pallas_call(
        _table_body,
        out_shape=jax.ShapeDtypeStruct((TAB, D), jnp.float32),
    )(token_embedding[:NB], pe)
    out = _sc_gather(x.reshape(-1), tab)
    return out.reshape(BATCH, SEQ, D)


# speedup vs baseline: 9.9442x; 1.0130x over previous
import functools

import jax
import jax.numpy as jnp
from jax import lax
from jax.experimental import pallas as pl
from jax.experimental.pallas import tpu as pltpu
from jax.experimental.pallas import tpu_sc as plsc

D = 256
HALF = 128
NB = 8
BATCH = 4096
SEQ = 64
NROWS = BATCH * SEQ
TAB = NB * NB * NB

_info = plsc.get_sparse_core_info()
_NC, _NS = _info.num_cores, _info.num_subcores
_NW = _NC * _NS
_BPW = BATCH // _NW
_CHUNK_B = 2
_CHUNK_R = _CHUNK_B * SEQ
_NCHUNK = _BPW // _CHUNK_B
_XPW = _BPW * 3 * SEQ


def _table_body(te_ref, pe_ref, c_ref):
    te8 = te_ref[...]
    pe = pe_ref[...]
    first = jnp.broadcast_to(pe[:, None, :], (NB, NB, HALF)).reshape(NB * NB, HALF)
    second = jnp.broadcast_to(pe[None, :, :], (NB, NB, HALF)).reshape(NB * NB, HALF)
    pos = jnp.concatenate([first, second], axis=-1)
    tok = jnp.broadcast_to(te8[:, None, :] * 16.0, (NB, NB * NB, D)).reshape(TAB, D)
    posr = jnp.broadcast_to(pos[None, :, :], (NB, NB * NB, D)).reshape(TAB, D)
    c_ref[...] = tok + posr


_mesh = plsc.VectorSubcoreMesh(core_axis_name="c", subcore_axis_name="s")


@functools.partial(
    pl.kernel,
    mesh=_mesh,
    out_type=jax.ShapeDtypeStruct((NROWS, D), jnp.float32),
    scratch_types=[
        pltpu.VMEM((_XPW,), jnp.int32),
        pltpu.VMEM((2, _CHUNK_R), jnp.int32),
        pltpu.VMEM((_CHUNK_R, D), jnp.float32),
        pltpu.VMEM((_CHUNK_R, D), jnp.float32),
        pltpu.SemaphoreType.DMA,
        pltpu.SemaphoreType.DMA,
        pltpu.SemaphoreType.DMA,
    ],
)
def _sc_gather(xf_hbm, tab_hbm, out_hbm, xv, idxv, rowsA, rowsB,
               gsem, wsA, wsB):
    wid = lax.axis_index("s") * _NC + lax.axis_index("c")
    pltpu.sync_copy(xf_hbm.at[pl.ds(wid * _XPW, _XPW)], xv)
    rbase = wid * (_BPW * SEQ)

    rows = (rowsA, rowsB)
    wsem = (wsA, wsB)

    def do_chunk(c, p):
        @pl.when(c >= 2)
        def _():
            pltpu.make_async_copy(
                rows[p], out_hbm.at[pl.ds(rbase, _CHUNK_R)], wsem[p]).wait()

        for j in range(_CHUNK_B):
            boff = (c * _CHUNK_B + j) * (3 * SEQ)
            for k in range(SEQ // 16):
                t = xv[pl.ds(boff + k * 16, 16)]
                p1 = xv[pl.ds(boff + SEQ + k * 16, 16)]
                p2 = xv[pl.ds(boff + 2 * SEQ + k * 16, 16)]
                idxv[p, pl.ds(j * SEQ + k * 16, 16)] = t * (NB * NB) + p1 * NB + p2
        pltpu.async_copy(tab_hbm.at[idxv.at[p]], rows[p], gsem).wait()
        pltpu.async_copy(
            rows[p], out_hbm.at[pl.ds(rbase + c * _CHUNK_R, _CHUNK_R)], wsem[p])

    def body(i, carry):
        do_chunk(2 * i, 0)
        do_chunk(2 * i + 1, 1)
        return carry

    lax.fori_loop(0, _NCHUNK // 2, body, 0)
    for p in range(2):
        pltpu.make_async_copy(
            rows[p], out_hbm.at[pl.ds(rbase, _CHUNK_R)], wsem[p]).wait()


def kernel(x, token_embedding, pe):
    tab = pl.pallas_call(
        _table_body,
        out_shape=jax.ShapeDtypeStruct((TAB, D), jnp.float32),
    )(token_embedding[:NB], pe)
    out = _sc_gather(x.reshape(-1), tab)
    return out.reshape(BATCH, SEQ, D)
